# baseline (speedup 1.0000x reference)
import functools

import jax
import jax.numpy as jnp
from jax import lax
from jax.experimental import pallas as pl
from jax.experimental.pallas import tpu as pltpu
from jax.experimental.pallas import tpu_sc as plsc

_NC = 2
_NS = 16
_NW = _NC * _NS

_VOCAB = 200
_DIM = 32
_LANES = 16
_BATCH = 16384
_SEQ = 200
_ROWS_PER_W = _BATCH // _NW
_CHUNK_B = 4
_NSLOTS = 4
_N_CHUNKS = _ROWS_PER_W // _CHUNK_B
_N_ITERS = _N_CHUNKS // _NSLOTS
_FULL_GROUPS = _SEQ // _LANES
_TAIL_START = _SEQ - _LANES

_mesh = plsc.VectorSubcoreMesh(
    core_axis_name="c", subcore_axis_name="s", num_cores=_NC, num_subcores=_NS
)


@functools.partial(
    pl.kernel,
    out_type=jax.ShapeDtypeStruct((_BATCH, _SEQ, _DIM), jnp.float32),
    mesh=_mesh,
    scratch_types=[
        pltpu.VMEM((_VOCAB, _DIM), jnp.float32),
        pltpu.VMEM((_NSLOTS, _CHUNK_B, _SEQ), jnp.int32),
        pltpu.VMEM((_NSLOTS, _CHUNK_B, _SEQ, _DIM), jnp.float32),
        pltpu.SemaphoreType.DMA,
        pltpu.SemaphoreType.DMA,
    ],
    compiler_params=pltpu.CompilerParams(
        use_tc_tiling_on_sc=False, needs_layout_passes=False
    ),
)
def _emb_lookup(pos_hbm, table_hbm, out_hbm, tab_v, idx_v, rows_v, sem_i, sem_o):
    wid = lax.axis_index("s") * _NC + lax.axis_index("c")
    base = wid * _ROWS_PER_W

    pltpu.sync_copy(table_hbm, tab_v)

    iota = lax.iota(jnp.int32, _LANES)
    cols = [(iota + c) & (_DIM - 1) for c in range(_DIM)]

    def start_idx(i, s):
        pltpu.async_copy(
            pos_hbm.at[pl.ds(base + i * _CHUNK_B, _CHUNK_B), :],
            idx_v.at[s],
            sem_i,
        )

    def wait_idx(s):
        pltpu.make_async_copy(
            pos_hbm.at[pl.ds(base, _CHUNK_B), :], idx_v.at[s], sem_i
        ).wait()

    def start_out(i, s):
        pltpu.async_copy(
            rows_v.at[s],
            out_hbm.at[pl.ds(base + i * _CHUNK_B, _CHUNK_B), :, :],
            sem_o,
        )

    def wait_out(s):
        pltpu.make_async_copy(
            rows_v.at[s], out_hbm.at[pl.ds(base, _CHUNK_B), :, :], sem_o
        ).wait()

    def expand_group(idx_row, rows_row, q0):
        iv = idx_row[pl.ds(q0, _LANES)]
        rowq = iota + q0
        lag = 4
        vs = [None] * _DIM
        for c in range(_DIM):
            vs[c] = plsc.load_gather(tab_v, [iv, cols[c]])
            if c >= lag:
                plsc.store_scatter(rows_row, [rowq, cols[c - lag]], vs[c - lag])
        for c in range(_DIM - lag, _DIM):
            plsc.store_scatter(rows_row, [rowq, cols[c]], vs[c])

    for s in range(_NSLOTS):
        start_idx(s, s)

    @pl.loop(0, _N_ITERS)
    def _ring(j):
        for s in range(_NSLOTS):
            i = j * _NSLOTS + s
            wait_idx(s)

            @pl.when(j > 0)
            def _():
                wait_out(s)

            for b in range(_CHUNK_B):
                idx_row = idx_v.at[s, b]
                rows_row = rows_v.at[s, b]

                @pl.loop(0, _FULL_GROUPS)
                def _group(k):
                    expand_group(idx_row, rows_row, k * _LANES)

                expand_group(idx_row, rows_row, _TAIL_START)

            @pl.when(j < _N_ITERS - 1)
            def _():
                start_idx(i + _NSLOTS, s)

            start_out(i, s)

    for s in range(_NSLOTS):
        wait_out(s)


def kernel(positions, table):
    return _emb_lookup(positions, table)


# speedup vs baseline: 1.2206x; 1.2206x over previous
import functools

import jax
import jax.numpy as jnp
from jax import lax
from jax.experimental import pallas as pl
from jax.experimental.pallas import tpu as pltpu
from jax.experimental.pallas import tpu_sc as plsc

_NC = 2
_NS = 16
_NW = _NC * _NS

_VOCAB = 200
_DIM = 32
_LANES = 16
_BATCH = 16384
_SEQ = 200
_ROWS_PER_W = _BATCH // _NW
_NSLOTS = 4
_FULL_GROUPS = _SEQ // _LANES
_TAIL_START = _SEQ - _LANES

_mesh = plsc.VectorSubcoreMesh(
    core_axis_name="c", subcore_axis_name="s", num_cores=_NC, num_subcores=_NS
)


@functools.partial(
    pl.kernel,
    out_type=jax.ShapeDtypeStruct((_BATCH, _SEQ, _DIM), jnp.float32),
    mesh=_mesh,
    scratch_types=[
        pltpu.VMEM((_VOCAB * _DIM,), jnp.float32),
        pltpu.VMEM((_NSLOTS, _SEQ), jnp.int32),
        pltpu.VMEM((_NSLOTS, _SEQ, _DIM), jnp.float32),
        pltpu.SemaphoreType.DMA,
        pltpu.SemaphoreType.DMA,
    ],
    compiler_params=pltpu.CompilerParams(
        use_tc_tiling_on_sc=True, needs_layout_passes=False
    ),
)
def _emb_lookup(pos_hbm, table_hbm, out_hbm, tab_v, idx_v, rows_v, sem_i, sem_o):
    wid = lax.axis_index("s") * _NC + lax.axis_index("c")
    base = wid * _ROWS_PER_W

    pltpu.sync_copy(table_hbm, tab_v)

    iota = lax.iota(jnp.int32, _LANES)
    zero = iota * 0
    cols = [(iota + c) & (_DIM - 1) for c in range(_DIM)]

    def start_idx(i, s):
        pltpu.async_copy(
            pos_hbm.at[pl.ds(base + i, 1), :], idx_v.at[pl.ds(s, 1)], sem_i
        )

    def wait_idx(s):
        pltpu.make_async_copy(
            pos_hbm.at[pl.ds(base, 1), :], idx_v.at[pl.ds(s, 1)], sem_i
        ).wait()

    def start_out(i, s):
        pltpu.async_copy(
            rows_v.at[pl.ds(s, 1)],
            out_hbm.at[pl.ds(base + i, 1), :, :],
            sem_o,
        )

    def wait_out(s):
        pltpu.make_async_copy(
            rows_v.at[pl.ds(s, 1)], out_hbm.at[pl.ds(base, 1), :, :], sem_o
        ).wait()

    def expand_group(s, q0):
        iv = idx_v[s, pl.ds(q0, _LANES)]
        iv32 = iv * _DIM
        rows_slot = rows_v.at[pl.ds(s, 1)]
        rowq = iota + q0
        lag = 4
        vs = [None] * _DIM
        for c in range(_DIM):
            vs[c] = plsc.load_gather(tab_v, [iv32 + cols[c]])
            if c >= lag:
                plsc.store_scatter(
                    rows_slot, [zero, rowq, cols[c - lag]], vs[c - lag]
                )
        for c in range(_DIM - lag, _DIM):
            plsc.store_scatter(rows_slot, [zero, rowq, cols[c]], vs[c])

    @pl.loop(0, _NSLOTS)
    def _prime(s):
        start_idx(s, s)

    @pl.loop(0, _ROWS_PER_W)
    def _row(i):
        s = lax.rem(i, _NSLOTS)
        wait_idx(s)

        @pl.when(i >= _NSLOTS)
        def _():
            wait_out(s)

        @pl.loop(0, _FULL_GROUPS)
        def _group(k):
            expand_group(s, k * _LANES)

        expand_group(s, _TAIL_START)

        @pl.when(i < _ROWS_PER_W - _NSLOTS)
        def _():
            start_idx(i + _NSLOTS, s)

        start_out(i, s)

    @pl.loop(0, _NSLOTS)
    def _drain(s):
        wait_out(s)


def kernel(positions, table):
    return _emb_lookup(positions, table.reshape(_VOCAB * _DIM))
